# Initial kernel scaffold; baseline (speedup 1.0000x reference)
import jax
import jax.numpy as jnp
from jax.experimental import pallas as pl


def kernel(idx, embedding):
    raise NotImplementedError("write your pallas kernel here")


# speedup vs baseline: 1.9735x; 1.9735x over previous
import functools

import jax
import jax.numpy as jnp
from jax import lax
from jax.experimental import pallas as pl
from jax.experimental.pallas import tpu as pltpu
from jax.experimental.pallas import tpu_sc as plsc

VOCAB = 8192
D = 8192
N = 8192
NC, NS = 2, 16
NW = NC * NS
TPW = N // NW
R = 4
STEPS = TPW // R
NBUF = 2


def _body(table_hbm, idx_hbm, out_hbm, idx_v, buf0, buf1, sem0, sem1):
    wid = lax.axis_index("s") * NC + lax.axis_index("c")
    base = wid * TPW

    pltpu.sync_copy(idx_hbm.at[wid], idx_v)

    bufs = (buf0, buf1)
    sems = (sem0, sem1)

    def start_gather(s, b):
        pltpu.make_async_copy(table_hbm.at[idx_v.at[s]], bufs[b], sems[b]).start()

    def wait_gather(b):
        pltpu.make_async_copy(table_hbm.at[idx_v.at[0]], bufs[b], sems[b]).wait()

    def put(s, b):
        pltpu.sync_copy(bufs[b], out_hbm.at[pl.ds(base + s * R, R)])

    for b in range(NBUF):
        start_gather(b, b)

    def outer(g, carry):
        for b in range(NBUF):
            s = g * NBUF + b
            wait_gather(b)
            put(s, b)
            start_gather(s + NBUF, b)
        return carry

    lax.fori_loop(0, STEPS // NBUF - 1, outer, 0)

    for b in range(NBUF):
        s = STEPS - NBUF + b
        wait_gather(b)
        put(s, b)


@functools.partial(jax.jit, static_argnames=())
def kernel(idx, embedding):
    B, L = idx.shape
    idx3 = idx.reshape(NW, STEPS, R).astype(jnp.int32)

    mesh = plsc.VectorSubcoreMesh(
        core_axis_name="c", subcore_axis_name="s", num_cores=NC, num_subcores=NS
    )
    out = pl.kernel(
        _body,
        out_type=jax.ShapeDtypeStruct((N, D), jnp.float32),
        mesh=mesh,
        scratch_types=[
            pltpu.VMEM((STEPS, R), jnp.int32),
            pltpu.VMEM((R, D), jnp.float32),
            pltpu.VMEM((R, D), jnp.float32),
            pltpu.SemaphoreType.DMA,
            pltpu.SemaphoreType.DMA,
        ],
    )(embedding, idx3)
    return out.reshape(B, L, D)
